# baseline (speedup 1.0000x reference)
import functools

import jax
import jax.numpy as jnp
from jax import lax
from jax.experimental import pallas as pl
from jax.experimental.pallas import tpu as pltpu
from jax.experimental.pallas import tpu_sc as plsc


def kernel(class_labels, table):
    (B,) = class_labels.shape
    V, D = table.shape
    idx = class_labels if class_labels.dtype == jnp.int32 else class_labels.astype(jnp.int32)

    info = plsc.get_sparse_core_info()
    NC, NS = info.num_cores, info.num_subcores
    NW = NC * NS
    b_per_w = B // NW
    assert B % (8 * NW) == 0

    CB = 64
    C = b_per_w // CB
    assert b_per_w % CB == 0
    WF = 2
    WB = WF * CB
    W = b_per_w // WB
    SCHUNK = 128
    n_full = V // SCHUNK
    rem = V - n_full * SCHUNK

    mesh = plsc.VectorSubcoreMesh(core_axis_name="c", subcore_axis_name="s")

    @functools.partial(
        pl.kernel,
        mesh=mesh,
        out_type=jax.ShapeDtypeStruct((B, D), jnp.float32),
        scratch_types=[
            pltpu.VMEM((b_per_w,), jnp.int32),
            pltpu.VMEM((b_per_w, D), jnp.float32),
            pltpu.VMEM_SHARED((V, D), jnp.float32),
            pltpu.SemaphoreType.DMA,
            pltpu.SemaphoreType.DMA,
        ],
    )
    def emb(table_hbm, idx_hbm, out_hbm, idx_v, rows_v, table_sp, gsem, osem):
        sid = lax.axis_index("s")
        wid = sid * NC + lax.axis_index("c")
        base = wid * b_per_w

        pltpu.sync_copy(idx_hbm.at[pl.ds(base, b_per_w)], idx_v)

        @pl.when(sid < n_full)
        def _():
            pltpu.sync_copy(
                table_hbm.at[pl.ds(sid * SCHUNK, SCHUNK)],
                table_sp.at[pl.ds(sid * SCHUNK, SCHUNK)],
            )

        if rem:
            @pl.when(sid == n_full)
            def _():
                pltpu.sync_copy(
                    table_hbm.at[pl.ds(n_full * SCHUNK, rem)],
                    table_sp.at[pl.ds(n_full * SCHUNK, rem)],
                )
        plsc.subcore_barrier()

        gathers = [
            pltpu.async_copy(
                table_sp.at[idx_v.at[pl.ds(g * CB, CB)]],
                rows_v.at[pl.ds(g * CB, CB)],
                gsem,
            )
            for g in range(C)
        ]
        outs = [None] * W
        for w in range(W):
            for g in range(w * WF, (w + 1) * WF):
                gathers[g].wait()
            outs[w] = pltpu.async_copy(
                rows_v.at[pl.ds(w * WB, WB)],
                out_hbm.at[pl.ds(base + w * WB, WB)],
                osem,
            )
        for w in range(W):
            outs[w].wait()

    return emb(table, idx)


# speedup vs baseline: 1.0065x; 1.0065x over previous
import functools

import jax
import jax.numpy as jnp
from jax import lax
from jax.experimental import pallas as pl
from jax.experimental.pallas import tpu as pltpu
from jax.experimental.pallas import tpu_sc as plsc


def kernel(class_labels, table):
    (B,) = class_labels.shape
    V, D = table.shape
    idx = class_labels if class_labels.dtype == jnp.int32 else class_labels.astype(jnp.int32)

    info = plsc.get_sparse_core_info()
    NC, NS = info.num_cores, info.num_subcores
    NW = NC * NS
    b_per_w = B // NW
    assert B % (8 * NW) == 0

    CB = 64
    C = b_per_w // CB
    assert b_per_w % CB == 0
    WF = 1
    WB = WF * CB
    W = b_per_w // WB
    SCHUNK = 128
    n_full = V // SCHUNK
    rem = V - n_full * SCHUNK

    mesh = plsc.VectorSubcoreMesh(core_axis_name="c", subcore_axis_name="s")

    @functools.partial(
        pl.kernel,
        mesh=mesh,
        out_type=jax.ShapeDtypeStruct((B, D), jnp.float32),
        scratch_types=[
            pltpu.VMEM((b_per_w,), jnp.int32),
            pltpu.VMEM((b_per_w, D), jnp.float32),
            pltpu.VMEM_SHARED((V, D), jnp.float32),
            pltpu.SemaphoreType.DMA,
            pltpu.SemaphoreType.DMA,
        ],
    )
    def emb(table_hbm, idx_hbm, out_hbm, idx_v, rows_v, table_sp, gsem, osem):
        sid = lax.axis_index("s")
        wid = sid * NC + lax.axis_index("c")
        base = wid * b_per_w

        pltpu.sync_copy(idx_hbm.at[pl.ds(base, b_per_w)], idx_v)

        @pl.when(sid < n_full)
        def _():
            pltpu.sync_copy(
                table_hbm.at[pl.ds(sid * SCHUNK, SCHUNK)],
                table_sp.at[pl.ds(sid * SCHUNK, SCHUNK)],
            )

        if rem:
            @pl.when(sid == n_full)
            def _():
                pltpu.sync_copy(
                    table_hbm.at[pl.ds(n_full * SCHUNK, rem)],
                    table_sp.at[pl.ds(n_full * SCHUNK, rem)],
                )
        plsc.subcore_barrier()

        gathers = [
            pltpu.async_copy(
                table_sp.at[idx_v.at[pl.ds(g * CB, CB)]],
                rows_v.at[pl.ds(g * CB, CB)],
                gsem,
            )
            for g in range(C)
        ]
        outs = [None] * W
        for w in range(W):
            for g in range(w * WF, (w + 1) * WF):
                gathers[g].wait()
            outs[w] = pltpu.async_copy(
                rows_v.at[pl.ds(w * WB, WB)],
                out_hbm.at[pl.ds(base + w * WB, WB)],
                osem,
            )
        for w in range(W):
            outs[w].wait()

    return emb(table, idx)


# speedup vs baseline: 1.0262x; 1.0196x over previous
import functools

import jax
import jax.numpy as jnp
from jax import lax
from jax.experimental import pallas as pl
from jax.experimental.pallas import tpu as pltpu
from jax.experimental.pallas import tpu_sc as plsc


def kernel(class_labels, table):
    (B,) = class_labels.shape
    V, D = table.shape
    idx = class_labels if class_labels.dtype == jnp.int32 else class_labels.astype(jnp.int32)

    info = plsc.get_sparse_core_info()
    NC, NS = info.num_cores, info.num_subcores
    NW = NC * NS
    b_per_w = B // NW
    assert B % (8 * NW) == 0

    CB = 64
    C = b_per_w // CB
    assert b_per_w % CB == 0
    WF = 1
    WB = WF * CB
    W = b_per_w // WB
    SCHUNK = 128
    n_full = V // SCHUNK
    rem = V - n_full * SCHUNK

    mesh = plsc.VectorSubcoreMesh(core_axis_name="c", subcore_axis_name="s")

    @functools.partial(
        pl.kernel,
        mesh=mesh,
        out_type=jax.ShapeDtypeStruct((B, D), jnp.float32),
        scratch_types=[
            pltpu.VMEM((b_per_w,), jnp.int32),
            pltpu.VMEM((b_per_w, D), jnp.float32),
            pltpu.VMEM_SHARED((V, D), jnp.float32),
            pltpu.SemaphoreType.DMA,
            pltpu.SemaphoreType.DMA,
            pltpu.SemaphoreType.DMA,
        ],
    )
    def emb(table_hbm, idx_hbm, out_hbm, idx_v, rows_v, table_sp, gsem, osem, isem):
        sid = lax.axis_index("s")
        wid = sid * NC + lax.axis_index("c")
        base = wid * b_per_w

        idx_cp = pltpu.async_copy(idx_hbm.at[pl.ds(base, b_per_w)], idx_v, isem)

        @pl.when(sid < n_full)
        def _():
            pltpu.sync_copy(
                table_hbm.at[pl.ds(sid * SCHUNK, SCHUNK)],
                table_sp.at[pl.ds(sid * SCHUNK, SCHUNK)],
            )

        if rem:
            @pl.when(sid == n_full)
            def _():
                pltpu.sync_copy(
                    table_hbm.at[pl.ds(n_full * SCHUNK, rem)],
                    table_sp.at[pl.ds(n_full * SCHUNK, rem)],
                )
        idx_cp.wait()
        plsc.subcore_barrier()

        gathers = [
            pltpu.async_copy(
                table_sp.at[idx_v.at[pl.ds(g * CB, CB)]],
                rows_v.at[pl.ds(g * CB, CB)],
                gsem,
            )
            for g in range(C)
        ]
        outs = [None] * W
        for w in range(W):
            for g in range(w * WF, (w + 1) * WF):
                gathers[g].wait()
            outs[w] = pltpu.async_copy(
                rows_v.at[pl.ds(w * WB, WB)],
                out_hbm.at[pl.ds(base + w * WB, WB)],
                osem,
            )
        for w in range(W):
            outs[w].wait()

    return emb(table, idx)
